# baseline (speedup 1.0000x reference)
import functools

import jax
import jax.numpy as jnp
import numpy as np
from jax import lax
from jax.experimental import pallas as pl
from jax.experimental.pallas import tpu as pltpu
from jax.experimental.pallas import tpu_sc as plsc

F32 = jnp.float32

N = 10000
E = 320000
H = 128
HID = 64

NC = 2
NS = 16
NW = NC * NS
EPW = E // NW
CHUNK = 80
NCHUNK = EPW // CHUNK
RPT = N // NS
ZR = 125
LANES = 16


def _sc_tanh(v):
  vc = jnp.minimum(jnp.maximum(v, -15.0), 15.0)
  e = jnp.exp(vc * 2.0)
  return (e - 1.0) / (e + 1.0)


NSLOT = 4
GDIST = 3
IDIST = 4

def _mk_perm():
  cmap = np.zeros(HID, np.int32)
  for p in range(HID):
    j, m = p // 32, p % 32
    cmap[p] = 32 * j + 2 * m if m < 16 else 32 * j + 2 * (m - 16) + 1
  inv = np.zeros(HID, np.int32)
  inv[cmap] = np.arange(HID, dtype=np.int32)
  return inv

_INV = _mk_perm()


def _pack16(t):
  return t[:, _INV].astype(jnp.bfloat16)


def _make_sc_edge_pass(with_cnt):
  mesh = plsc.VectorSubcoreMesh(core_axis_name="c", subcore_axis_name="s")

  out_type = [jax.ShapeDtypeStruct((NC, N, HID), F32)]
  if with_cnt:
    out_type.append(jax.ShapeDtypeStruct((NC, N, LANES), F32))

  scratch = dict(
      idx_d=[pltpu.VMEM((CHUNK,), jnp.int32) for _ in range(NSLOT)],
      idx_s=[pltpu.VMEM((CHUNK,), jnp.int32) for _ in range(NSLOT)],
      sidx=[pltpu.VMEM((CHUNK,), jnp.int32) for _ in range(NSLOT)],
      rows_d=[pltpu.VMEM((CHUNK, HID), jnp.bfloat16)
              for _ in range(NSLOT)],
      rows_s=[pltpu.VMEM((CHUNK, HID), jnp.bfloat16)
              for _ in range(NSLOT)],
      tbuf=[pltpu.VMEM((CHUNK, HID), F32) for _ in range(NSLOT)],
      zbuf_s=pltpu.VMEM((ZR, HID), F32),
      sh_s=pltpu.VMEM_SHARED((N, HID), F32),
      sem_ixd=[pltpu.SemaphoreType.DMA for _ in range(NSLOT)],
      sem_ixs=[pltpu.SemaphoreType.DMA for _ in range(NSLOT)],
      sem_gd=[pltpu.SemaphoreType.DMA for _ in range(NSLOT)],
      sem_gs=[pltpu.SemaphoreType.DMA for _ in range(NSLOT)],
      sem_sc=[pltpu.SemaphoreType.DMA for _ in range(NSLOT)],
  )
  if with_cnt:
    scratch.update(
        ones=pltpu.VMEM((CHUNK, LANES), F32),
        zbuf_c=pltpu.VMEM((ZR, LANES), F32),
        sh_c=pltpu.VMEM_SHARED((N, LANES), F32),
        sem_sc2=[pltpu.SemaphoreType.DMA for _ in range(NSLOT)],
    )

  def body(ei_h, pd_h, ps_h, out_s, *rest):
    if with_cnt:
      (out_c, idx_d, idx_s, sidx, rows_d, rows_s, tbuf, zbuf_s, sh_s,
       sem_ixd, sem_ixs, sem_gd, sem_gs, sem_sc, ones, zbuf_c, sh_c,
       sem_sc2) = rest
    else:
      (idx_d, idx_s, sidx, rows_d, rows_s, tbuf, zbuf_s, sh_s, sem_ixd,
       sem_ixs, sem_gd, sem_gs, sem_sc) = rest

    cid = lax.axis_index("c")
    sid = lax.axis_index("s")
    wid = sid * NC + cid
    ebase = wid * EPW

    zero = jnp.zeros((LANES,), F32)

    def zfill(r, _):
      for j in range(HID // LANES):
        zbuf_s[r, pl.ds(j * LANES, LANES)] = zero
      if with_cnt:
        zbuf_c[r, pl.ds(0, LANES)] = zero
      return 0
    lax.fori_loop(0, ZR, zfill, 0)
    if with_cnt:
      lane_iota = lax.iota(jnp.int32, LANES)
      one0 = jnp.where(lane_iota == 0, 1.0, 0.0).astype(F32)
      def ofill(r, _):
        ones[r, pl.ds(0, LANES)] = one0
        return 0
      lax.fori_loop(0, CHUNK, ofill, 0)
    for k in range(RPT // ZR):
      roff = sid * RPT + k * ZR
      pltpu.sync_copy(zbuf_s, sh_s.at[pl.ds(roff, ZR), :])
      if with_cnt:
        pltpu.sync_copy(zbuf_c, sh_c.at[pl.ds(roff, ZR), :])
    plsc.subcore_barrier()

    def idx_copies(c, s):
      off = ebase + c * CHUNK
      return (
          pltpu.make_async_copy(
              ei_h.at[1, pl.ds(off, CHUNK)], idx_d[s], sem_ixd[s]),
          pltpu.make_async_copy(
              ei_h.at[0, pl.ds(off, CHUNK)], idx_s[s], sem_ixs[s]),
      )

    def gathers(c, s):
      cpd = pltpu.make_async_copy(pd_h.at[idx_d[s]], rows_d[s], sem_gd[s])
      cps = pltpu.make_async_copy(ps_h.at[idx_s[s]], rows_s[s], sem_gs[s])
      return cpd, cps

    def scatters_start(s):
      pltpu.async_copy(tbuf[s], sh_s.at[sidx[s]], sem_sc[s], add=True)
      if with_cnt:
        pltpu.async_copy(ones, sh_c.at[sidx[s]], sem_sc2[s], add=True)

    def scatters(s):
      out = [pltpu.make_async_copy(tbuf[s], sh_s.at[sidx[s]], sem_sc[s])]
      if with_cnt:
        out.append(
            pltpu.make_async_copy(ones, sh_c.at[sidx[s]], sem_sc2[s]))
      return out

    for s in range(NSLOT):
      for cp in idx_copies(s, s):
        cp.start()
    for s in range(GDIST):
      for cp in idx_copies(s, s):
        cp.wait()
      for cp in gathers(s, s):
        cp.start()

    def step(c, s, wait_scat, pre_idx, issue_gather):
      for cp in gathers(c, s):
        cp.wait()
      if wait_scat:
        for cp in scatters(s):
          cp.wait()
      for j in range(CHUNK // LANES):
        sidx[s][pl.ds(j * LANES, LANES)] = idx_d[s][pl.ds(j * LANES, LANES)]
      if pre_idx:
        for cp in idx_copies(c + IDIST, s):
          cp.start()
      @plsc.parallel_loop(0, CHUNK)
      def row_body(r):
        for j in range(HID // 32):
          wd = rows_d[s][r, pl.ds(j * 32, 32)]
          ws = rows_s[s][r, pl.ds(j * 32, 32)]
          da, db = plsc.unpack(wd, format=plsc.PackFormat.INTERLEAVED)
          sa, sb = plsc.unpack(ws, format=plsc.PackFormat.INTERLEAVED)
          tbuf[s][r, pl.ds(j * 32, LANES)] = _sc_tanh(da + sa)
          tbuf[s][r, pl.ds(j * 32 + LANES, LANES)] = _sc_tanh(db + sb)
      scatters_start(s)
      if issue_gather:
        s2 = (s + GDIST) % NSLOT
        c2 = c + GDIST
        for cp in idx_copies(c2, s2):
          cp.wait()
        for cp in gathers(c2, s2):
          cp.start()

    for s in range(NSLOT):
      step(s, s, False, True, True)

    @pl.loop(1, NCHUNK // NSLOT - 1)
    def _(i):
      c0 = i * NSLOT
      for s in range(NSLOT):
        step(c0 + s, s, True, True, True)

    c0 = (NCHUNK // NSLOT - 1) * NSLOT
    for s in range(NSLOT):
      c = c0 + s
      step(c, s, True, c + IDIST < NCHUNK, c + GDIST < NCHUNK)
    step(NCHUNK - 1, (NCHUNK - 1) % NSLOT, True, False, False)
    for s in range(NSLOT):
      for cp in scatters(s):
        cp.wait()

    plsc.subcore_barrier()

    roff = sid * RPT
    pltpu.sync_copy(sh_s.at[pl.ds(roff, RPT), :],
                    out_s.at[cid, pl.ds(roff, RPT), :])
    if with_cnt:
      pltpu.sync_copy(sh_c.at[pl.ds(roff, RPT), :],
                      out_c.at[cid, pl.ds(roff, RPT), :])

  fn = pl.kernel(
      body, out_type=out_type, mesh=mesh,
      scratch_types=list(scratch.values()),
      compiler_params=pltpu.CompilerParams(
          use_tc_tiling_on_sc=False, needs_layout_passes=False),
  )
  return fn


_sc_pass_cnt = None
_sc_pass_nocnt = None


def _get_sc_passes():
  global _sc_pass_cnt, _sc_pass_nocnt
  if _sc_pass_cnt is None:
    _sc_pass_cnt = _make_sc_edge_pass(True)
    _sc_pass_nocnt = _make_sc_edge_pass(False)
  return _sc_pass_cnt, _sc_pass_nocnt


BLK = 1000
GRID = N // BLK


def _dot(a, b):
  return lax.dot_general(a, b, (((1,), (0,)), ((), ())),
                         preferred_element_type=F32)


def _stage1_body(x_ref, encW, encb, w1a, w1b, b1, h_ref, pd_ref, ps_ref):
  h = jnp.tanh(_dot(x_ref[...], encW[...]) + encb[...])
  h_ref[...] = h
  pd_ref[...] = _dot(h, w1a[...]) + b1[...]
  ps_ref[...] = _dot(h, w1b[...])


def _gru_update(h, s0, s1, c0, c1, pd, ps, w2, b2, wih, whh, bih, bhh):
  tself = jnp.tanh(pd + ps)
  s = s0 + s1 + tself
  cnt = c0[:, :1] + c1[:, :1] + 1.0
  agg = _dot(s / cnt, w2) + b2
  gi = _dot(agg, wih) + bih
  gh = _dot(h, whh) + bhh
  r = jax.nn.sigmoid(gi[:, :H] + gh[:, :H])
  z = jax.nn.sigmoid(gi[:, H:2 * H] + gh[:, H:2 * H])
  n = jnp.tanh(gi[:, 2 * H:] + r * gh[:, 2 * H:])
  return (1.0 - z) * n + z * h


def _stage2_body(h_ref, pd_ref, ps_ref, sp, cp, w2, b2, wih, whh,
                 bih, bhh, w1a, w1b, b1, hn_ref, pdn_ref, psn_ref):
  hn = _gru_update(h_ref[...], sp[0], sp[1], cp[0], cp[1],
                   pd_ref[...], ps_ref[...], w2[...], b2[...], wih[...],
                   whh[...], bih[...], bhh[...])
  hn_ref[...] = hn
  pdn_ref[...] = _dot(hn, w1a[...]) + b1[...]
  psn_ref[...] = _dot(hn, w1b[...])


def _stage3_body(h_ref, pd_ref, ps_ref, sp, cp, w2, b2, wih, whh,
                 bih, bhh, dw1, db1, dw2, db2, out_ref):
  hn = _gru_update(h_ref[...], sp[0], sp[1], cp[0], cp[1],
                   pd_ref[...], ps_ref[...], w2[...], b2[...], wih[...],
                   whh[...], bih[...], bhh[...])
  out_ref[...] = _dot(jnp.tanh(_dot(hn, dw1[...]) + db1[...]), dw2[...]) \
      + db2[...]


def _row_spec(width):
  return pl.BlockSpec((BLK, width), lambda i: (i, 0))


def _part_spec(width):
  return pl.BlockSpec((NC, BLK, width), lambda i: (0, i, 0))


def _full_spec(shape):
  return pl.BlockSpec(shape, lambda i: tuple(0 for _ in shape))


def _tc_call(body, in_specs, out_widths, args):
  out_shape = [jax.ShapeDtypeStruct((N, w), F32) for w in out_widths]
  return pl.pallas_call(
      body,
      grid=(GRID,),
      in_specs=in_specs,
      out_specs=[_row_spec(w) for w in out_widths],
      out_shape=out_shape,
  )(*args)


def kernel(x, edge_index, enc_W, enc_b, msg_W1_0, msg_b1_0, msg_W2_0,
           msg_b2_0, gru_Wih_0, gru_Whh_0, gru_bih_0, gru_bhh_0, msg_W1_1,
           msg_b1_1, msg_W2_1, msg_b2_1, gru_Wih_1, gru_Whh_1, gru_bih_1,
           gru_bhh_1, dec_W1, dec_b1, dec_W2, dec_b2):
  enc_b2d = enc_b.reshape(1, H)
  w1a_0, w1b_0 = msg_W1_0[:H], msg_W1_0[H:]
  w1a_1, w1b_1 = msg_W1_1[:H], msg_W1_1[H:]
  b1_0 = msg_b1_0.reshape(1, HID)
  b1_1 = msg_b1_1.reshape(1, HID)
  b2_0 = msg_b2_0.reshape(1, H)
  b2_1 = msg_b2_1.reshape(1, H)
  bih_0 = gru_bih_0.reshape(1, 3 * H)
  bhh_0 = gru_bhh_0.reshape(1, 3 * H)
  bih_1 = gru_bih_1.reshape(1, 3 * H)
  bhh_1 = gru_bhh_1.reshape(1, 3 * H)
  db1 = dec_b1.reshape(1, HID)
  db2 = dec_b2.reshape(1, 1)

  sc_cnt, sc_nocnt = _get_sc_passes()

  h0, pd0, ps0 = _tc_call(
      _stage1_body,
      [_row_spec(H), _full_spec((H, H)), _full_spec((1, H)),
       _full_spec((H, HID)), _full_spec((H, HID)), _full_spec((1, HID))],
      [H, HID, HID],
      [x, enc_W, enc_b2d, w1a_0, w1b_0, b1_0],
  )

  s_part0, c_part = sc_cnt(edge_index, _pack16(pd0), _pack16(ps0))

  gru_specs = [_full_spec((HID, H)), _full_spec((1, H)),
               _full_spec((H, 3 * H)), _full_spec((H, 3 * H)),
               _full_spec((1, 3 * H)), _full_spec((1, 3 * H))]

  h1, pd1, ps1 = _tc_call(
      _stage2_body,
      [_row_spec(H), _row_spec(HID), _row_spec(HID), _part_spec(HID),
       _part_spec(LANES)] + gru_specs +
      [_full_spec((H, HID)), _full_spec((H, HID)), _full_spec((1, HID))],
      [H, HID, HID],
      [h0, pd0, ps0, s_part0, c_part,
       msg_W2_0, b2_0, gru_Wih_0, gru_Whh_0, bih_0, bhh_0,
       w1a_1, w1b_1, b1_1],
  )

  (s_part1,) = sc_nocnt(edge_index, _pack16(pd1), _pack16(ps1))

  (out,) = _tc_call(
      _stage3_body,
      [_row_spec(H), _row_spec(HID), _row_spec(HID), _part_spec(HID),
       _part_spec(LANES)] + gru_specs +
      [_full_spec((H, HID)), _full_spec((1, HID)), _full_spec((HID, 1)),
       _full_spec((1, 1))],
      [1],
      [h1, pd1, ps1, s_part1, c_part,
       msg_W2_1, b2_1, gru_Wih_1, gru_Whh_1, bih_1, bhh_1,
       dec_W1, db1, dec_W2, db2],
  )

  return out.reshape(N)


# speedup vs baseline: 1.1355x; 1.1355x over previous
import functools

import jax
import jax.numpy as jnp
from jax import lax
from jax.experimental import pallas as pl
from jax.experimental.pallas import tpu as pltpu
from jax.experimental.pallas import tpu_sc as plsc

F32 = jnp.float32

N = 10000
E = 320000
H = 128
HID = 64

NC = 2
NS = 16
NW = NC * NS
EPW = E // NW
CHUNK = 80
NCHUNK = EPW // CHUNK
RPT = N // NS
ZR = 25
LANES = 16


def _sc_tanh(v):
  vc = jnp.minimum(jnp.maximum(v, -15.0), 15.0)
  e = jnp.exp(vc * 2.0)
  return (e - 1.0) / (e + 1.0)


NSLOT = 5
GDIST = 4
IDIST = 5


def _make_sc_edge_pass(with_cnt):
  mesh = plsc.VectorSubcoreMesh(core_axis_name="c", subcore_axis_name="s")

  out_type = [jax.ShapeDtypeStruct((NC, N, HID), F32)]
  if with_cnt:
    out_type.append(jax.ShapeDtypeStruct((NC, N, LANES), F32))

  scratch = dict(
      idx_d=[pltpu.VMEM((CHUNK,), jnp.int32) for _ in range(NSLOT)],
      idx_s=[pltpu.VMEM((CHUNK,), jnp.int32) for _ in range(NSLOT)],
      sidx=[pltpu.VMEM((CHUNK,), jnp.int32) for _ in range(NSLOT)],
      rows_d=[pltpu.VMEM((CHUNK, HID), F32) for _ in range(NSLOT)],
      rows_s=[pltpu.VMEM((CHUNK, HID), F32) for _ in range(NSLOT)],
      tbuf=[pltpu.VMEM((CHUNK, HID), F32) for _ in range(NSLOT)],
      zbuf_s=pltpu.VMEM((ZR, HID), F32),
      sh_s=pltpu.VMEM_SHARED((N, HID), F32),
      sem_ixd=[pltpu.SemaphoreType.DMA for _ in range(NSLOT)],
      sem_ixs=[pltpu.SemaphoreType.DMA for _ in range(NSLOT)],
      sem_gd=[pltpu.SemaphoreType.DMA for _ in range(NSLOT)],
      sem_gs=[pltpu.SemaphoreType.DMA for _ in range(NSLOT)],
      sem_sc=[pltpu.SemaphoreType.DMA for _ in range(NSLOT)],
  )
  if with_cnt:
    scratch.update(
        ones=pltpu.VMEM((CHUNK, LANES), F32),
        sh_c=pltpu.VMEM_SHARED((N, LANES), F32),
        sem_sc2=[pltpu.SemaphoreType.DMA for _ in range(NSLOT)],
    )

  def body(ei_h, pd_h, ps_h, out_s, *rest):
    if with_cnt:
      (out_c, idx_d, idx_s, sidx, rows_d, rows_s, tbuf, zbuf_s, sh_s,
       sem_ixd, sem_ixs, sem_gd, sem_gs, sem_sc, ones, sh_c,
       sem_sc2) = rest
    else:
      (idx_d, idx_s, sidx, rows_d, rows_s, tbuf, zbuf_s, sh_s, sem_ixd,
       sem_ixs, sem_gd, sem_gs, sem_sc) = rest

    cid = lax.axis_index("c")
    sid = lax.axis_index("s")
    wid = sid * NC + cid
    ebase = wid * EPW

    zero = jnp.zeros((LANES,), F32)

    def zfill(r, _):
      for j in range(HID // LANES):
        zbuf_s[r, pl.ds(j * LANES, LANES)] = zero
      return 0
    lax.fori_loop(0, ZR, zfill, 0)
    if with_cnt:
      lane_iota = lax.iota(jnp.int32, LANES)
      one0 = jnp.where(lane_iota == 0, 1.0, 0.0).astype(F32)
      def ofill(r, _):
        ones[r, pl.ds(0, LANES)] = one0
        return 0
      lax.fori_loop(0, CHUNK, ofill, 0)
    for k in range(RPT // ZR):
      roff = sid * RPT + k * ZR
      pltpu.sync_copy(zbuf_s, sh_s.at[pl.ds(roff, ZR), :])
      if with_cnt:
        pltpu.sync_copy(zbuf_s.at[pl.ds(0, ZR), pl.ds(0, LANES)],
                        sh_c.at[pl.ds(roff, ZR), :])
    plsc.subcore_barrier()

    def idx_copies(c, s):
      off = ebase + c * CHUNK
      return (
          pltpu.make_async_copy(
              ei_h.at[1, pl.ds(off, CHUNK)], idx_d[s], sem_ixd[s]),
          pltpu.make_async_copy(
              ei_h.at[0, pl.ds(off, CHUNK)], idx_s[s], sem_ixs[s]),
      )

    def gathers(c, s):
      cpd = pltpu.make_async_copy(pd_h.at[idx_d[s]], rows_d[s], sem_gd[s])
      cps = pltpu.make_async_copy(ps_h.at[idx_s[s]], rows_s[s], sem_gs[s])
      return cpd, cps

    def scatters_start(s):
      pltpu.async_copy(tbuf[s], sh_s.at[sidx[s]], sem_sc[s], add=True)
      if with_cnt:
        pltpu.async_copy(ones, sh_c.at[sidx[s]], sem_sc2[s], add=True)

    def scatters(s):
      out = [pltpu.make_async_copy(tbuf[s], sh_s.at[sidx[s]], sem_sc[s])]
      if with_cnt:
        out.append(
            pltpu.make_async_copy(ones, sh_c.at[sidx[s]], sem_sc2[s]))
      return out

    for s in range(NSLOT):
      for cp in idx_copies(s, s):
        cp.start()
    for s in range(GDIST):
      for cp in idx_copies(s, s):
        cp.wait()
      for cp in gathers(s, s):
        cp.start()

    def step(c, s, wait_scat, pre_idx, issue_gather):
      for cp in gathers(c, s):
        cp.wait()
      if wait_scat:
        for cp in scatters(s):
          cp.wait()
      for j in range(CHUNK // LANES):
        sidx[s][pl.ds(j * LANES, LANES)] = idx_d[s][pl.ds(j * LANES, LANES)]
      if pre_idx:
        for cp in idx_copies(c + IDIST, s):
          cp.start()
      @plsc.parallel_loop(0, CHUNK)
      def row_body(r):
        for j in range(HID // LANES):
          vd = rows_d[s][r, pl.ds(j * LANES, LANES)]
          vs = rows_s[s][r, pl.ds(j * LANES, LANES)]
          tbuf[s][r, pl.ds(j * LANES, LANES)] = _sc_tanh(vd + vs)
      scatters_start(s)
      if issue_gather:
        s2 = (s + GDIST) % NSLOT
        c2 = c + GDIST
        for cp in idx_copies(c2, s2):
          cp.wait()
        for cp in gathers(c2, s2):
          cp.start()

    for s in range(NSLOT):
      step(s, s, False, True, True)

    @pl.loop(1, NCHUNK // NSLOT - 1)
    def _(i):
      c0 = i * NSLOT
      for s in range(NSLOT):
        step(c0 + s, s, True, True, True)

    c0 = (NCHUNK // NSLOT - 1) * NSLOT
    for s in range(NSLOT):
      c = c0 + s
      step(c, s, True, c + IDIST < NCHUNK, c + GDIST < NCHUNK)
    for k in range(NCHUNK % NSLOT):
      c = (NCHUNK // NSLOT) * NSLOT + k
      step(c, c % NSLOT, True, False, False)
    for s in range(NSLOT):
      for cp in scatters(s):
        cp.wait()

    plsc.subcore_barrier()

    roff = sid * RPT
    pltpu.sync_copy(sh_s.at[pl.ds(roff, RPT), :],
                    out_s.at[cid, pl.ds(roff, RPT), :])
    if with_cnt:
      pltpu.sync_copy(sh_c.at[pl.ds(roff, RPT), :],
                      out_c.at[cid, pl.ds(roff, RPT), :])

  fn = pl.kernel(
      body, out_type=out_type, mesh=mesh,
      scratch_types=list(scratch.values()),
      compiler_params=pltpu.CompilerParams(use_tc_tiling_on_sc=False),
  )
  return fn


_sc_pass_cnt = None
_sc_pass_nocnt = None


def _get_sc_passes():
  global _sc_pass_cnt, _sc_pass_nocnt
  if _sc_pass_cnt is None:
    _sc_pass_cnt = _make_sc_edge_pass(True)
    _sc_pass_nocnt = _make_sc_edge_pass(False)
  return _sc_pass_cnt, _sc_pass_nocnt


BLK = 1000
GRID = N // BLK


def _dot(a, b):
  return lax.dot_general(a, b, (((1,), (0,)), ((), ())),
                         preferred_element_type=F32)


def _stage1_body(x_ref, encW, encb, w1a, w1b, b1, h_ref, pd_ref, ps_ref):
  h = jnp.tanh(_dot(x_ref[...], encW[...]) + encb[...])
  h_ref[...] = h
  pd_ref[...] = _dot(h, w1a[...]) + b1[...]
  ps_ref[...] = _dot(h, w1b[...])


def _gru_update(h, s0, s1, c0, c1, pd, ps, w2, b2, wih, whh, bih, bhh):
  tself = jnp.tanh(pd + ps)
  s = s0 + s1 + tself
  cnt = c0[:, :1] + c1[:, :1] + 1.0
  agg = _dot(s / cnt, w2) + b2
  gi = _dot(agg, wih) + bih
  gh = _dot(h, whh) + bhh
  r = jax.nn.sigmoid(gi[:, :H] + gh[:, :H])
  z = jax.nn.sigmoid(gi[:, H:2 * H] + gh[:, H:2 * H])
  n = jnp.tanh(gi[:, 2 * H:] + r * gh[:, 2 * H:])
  return (1.0 - z) * n + z * h


def _stage2_body(h_ref, pd_ref, ps_ref, sp, cp, w2, b2, wih, whh,
                 bih, bhh, w1a, w1b, b1, hn_ref, pdn_ref, psn_ref):
  hn = _gru_update(h_ref[...], sp[0], sp[1], cp[0], cp[1],
                   pd_ref[...], ps_ref[...], w2[...], b2[...], wih[...],
                   whh[...], bih[...], bhh[...])
  hn_ref[...] = hn
  pdn_ref[...] = _dot(hn, w1a[...]) + b1[...]
  psn_ref[...] = _dot(hn, w1b[...])


def _stage3_body(h_ref, pd_ref, ps_ref, sp, cp, w2, b2, wih, whh,
                 bih, bhh, dw1, db1, dw2, db2, out_ref):
  hn = _gru_update(h_ref[...], sp[0], sp[1], cp[0], cp[1],
                   pd_ref[...], ps_ref[...], w2[...], b2[...], wih[...],
                   whh[...], bih[...], bhh[...])
  out_ref[...] = _dot(jnp.tanh(_dot(hn, dw1[...]) + db1[...]), dw2[...]) \
      + db2[...]


def _row_spec(width):
  return pl.BlockSpec((BLK, width), lambda i: (i, 0))


def _part_spec(width):
  return pl.BlockSpec((NC, BLK, width), lambda i: (0, i, 0))


def _full_spec(shape):
  return pl.BlockSpec(shape, lambda i: tuple(0 for _ in shape))


def _tc_call(body, in_specs, out_widths, args):
  out_shape = [jax.ShapeDtypeStruct((N, w), F32) for w in out_widths]
  return pl.pallas_call(
      body,
      grid=(GRID,),
      in_specs=in_specs,
      out_specs=[_row_spec(w) for w in out_widths],
      out_shape=out_shape,
  )(*args)


def kernel(x, edge_index, enc_W, enc_b, msg_W1_0, msg_b1_0, msg_W2_0,
           msg_b2_0, gru_Wih_0, gru_Whh_0, gru_bih_0, gru_bhh_0, msg_W1_1,
           msg_b1_1, msg_W2_1, msg_b2_1, gru_Wih_1, gru_Whh_1, gru_bih_1,
           gru_bhh_1, dec_W1, dec_b1, dec_W2, dec_b2):
  enc_b2d = enc_b.reshape(1, H)
  w1a_0, w1b_0 = msg_W1_0[:H], msg_W1_0[H:]
  w1a_1, w1b_1 = msg_W1_1[:H], msg_W1_1[H:]
  b1_0 = msg_b1_0.reshape(1, HID)
  b1_1 = msg_b1_1.reshape(1, HID)
  b2_0 = msg_b2_0.reshape(1, H)
  b2_1 = msg_b2_1.reshape(1, H)
  bih_0 = gru_bih_0.reshape(1, 3 * H)
  bhh_0 = gru_bhh_0.reshape(1, 3 * H)
  bih_1 = gru_bih_1.reshape(1, 3 * H)
  bhh_1 = gru_bhh_1.reshape(1, 3 * H)
  db1 = dec_b1.reshape(1, HID)
  db2 = dec_b2.reshape(1, 1)

  sc_cnt, sc_nocnt = _get_sc_passes()

  h0, pd0, ps0 = _tc_call(
      _stage1_body,
      [_row_spec(H), _full_spec((H, H)), _full_spec((1, H)),
       _full_spec((H, HID)), _full_spec((H, HID)), _full_spec((1, HID))],
      [H, HID, HID],
      [x, enc_W, enc_b2d, w1a_0, w1b_0, b1_0],
  )

  s_part0, c_part = sc_cnt(edge_index, pd0, ps0)

  gru_specs = [_full_spec((HID, H)), _full_spec((1, H)),
               _full_spec((H, 3 * H)), _full_spec((H, 3 * H)),
               _full_spec((1, 3 * H)), _full_spec((1, 3 * H))]

  h1, pd1, ps1 = _tc_call(
      _stage2_body,
      [_row_spec(H), _row_spec(HID), _row_spec(HID), _part_spec(HID),
       _part_spec(LANES)] + gru_specs +
      [_full_spec((H, HID)), _full_spec((H, HID)), _full_spec((1, HID))],
      [H, HID, HID],
      [h0, pd0, ps0, s_part0, c_part,
       msg_W2_0, b2_0, gru_Wih_0, gru_Whh_0, bih_0, bhh_0,
       w1a_1, w1b_1, b1_1],
  )

  (s_part1,) = sc_nocnt(edge_index, pd1, ps1)

  (out,) = _tc_call(
      _stage3_body,
      [_row_spec(H), _row_spec(HID), _row_spec(HID), _part_spec(HID),
       _part_spec(LANES)] + gru_specs +
      [_full_spec((H, HID)), _full_spec((1, HID)), _full_spec((HID, 1)),
       _full_spec((1, 1))],
      [1],
      [h1, pd1, ps1, s_part1, c_part,
       msg_W2_1, b2_1, gru_Wih_1, gru_Whh_1, bih_1, bhh_1,
       dec_W1, db1, dec_W2, db2],
  )

  return out.reshape(N)


# speedup vs baseline: 1.1935x; 1.0511x over previous
import functools

import jax
import jax.numpy as jnp
from jax import lax
from jax.experimental import pallas as pl
from jax.experimental.pallas import tpu as pltpu
from jax.experimental.pallas import tpu_sc as plsc

F32 = jnp.float32

N = 10000
E = 320000
H = 128
HID = 64

NC = 2
NS = 16
NW = NC * NS
EPW = E // NW
CHUNK = 80
NCHUNK = EPW // CHUNK
RPT = N // NS
ZR = 125
LANES = 16


def _sc_tanh(v):
  vc = jnp.minimum(jnp.maximum(v, -15.0), 15.0)
  e = jnp.exp(vc * 2.0)
  return (e - 1.0) / (e + 1.0)


NSLOT = 4
GDIST = 3
IDIST = 4


def _make_sc_edge_pass(with_cnt):
  mesh = plsc.VectorSubcoreMesh(core_axis_name="c", subcore_axis_name="s")

  out_type = [jax.ShapeDtypeStruct((NC, N, HID), F32)]
  if with_cnt:
    out_type.append(jax.ShapeDtypeStruct((NC, N, LANES), F32))

  scratch = dict(
      idx_d=[pltpu.VMEM((CHUNK,), jnp.int32) for _ in range(NSLOT)],
      idx_s=[pltpu.VMEM((CHUNK,), jnp.int32) for _ in range(NSLOT)],
      sidx=[pltpu.VMEM((CHUNK,), jnp.int32) for _ in range(NSLOT)],
      rows_d=[pltpu.VMEM((CHUNK, HID), F32) for _ in range(NSLOT)],
      rows_s=[pltpu.VMEM((CHUNK, HID), F32) for _ in range(NSLOT)],
      tbuf=[pltpu.VMEM((CHUNK, HID), F32) for _ in range(NSLOT)],
      zbuf_s=pltpu.VMEM((ZR, HID), F32),
      sh_s=pltpu.VMEM_SHARED((N, HID), F32),
      sem_ixd=[pltpu.SemaphoreType.DMA for _ in range(NSLOT)],
      sem_ixs=[pltpu.SemaphoreType.DMA for _ in range(NSLOT)],
      sem_gd=[pltpu.SemaphoreType.DMA for _ in range(NSLOT)],
      sem_gs=[pltpu.SemaphoreType.DMA for _ in range(NSLOT)],
      sem_sc=[pltpu.SemaphoreType.DMA for _ in range(NSLOT)],
  )
  if with_cnt:
    scratch.update(
        ones=pltpu.VMEM((CHUNK, LANES), F32),
        zbuf_c=pltpu.VMEM((ZR, LANES), F32),
        sh_c=pltpu.VMEM_SHARED((N, LANES), F32),
        sem_sc2=[pltpu.SemaphoreType.DMA for _ in range(NSLOT)],
    )

  def body(ei_h, pd_h, ps_h, out_s, *rest):
    if with_cnt:
      (out_c, idx_d, idx_s, sidx, rows_d, rows_s, tbuf, zbuf_s, sh_s,
       sem_ixd, sem_ixs, sem_gd, sem_gs, sem_sc, ones, zbuf_c, sh_c,
       sem_sc2) = rest
    else:
      (idx_d, idx_s, sidx, rows_d, rows_s, tbuf, zbuf_s, sh_s, sem_ixd,
       sem_ixs, sem_gd, sem_gs, sem_sc) = rest

    cid = lax.axis_index("c")
    sid = lax.axis_index("s")
    wid = sid * NC + cid
    ebase = wid * EPW

    zero = jnp.zeros((LANES,), F32)

    def zfill(r, _):
      for j in range(HID // LANES):
        zbuf_s[r, pl.ds(j * LANES, LANES)] = zero
      if with_cnt:
        zbuf_c[r, pl.ds(0, LANES)] = zero
      return 0
    lax.fori_loop(0, ZR, zfill, 0)
    if with_cnt:
      lane_iota = lax.iota(jnp.int32, LANES)
      one0 = jnp.where(lane_iota == 0, 1.0, 0.0).astype(F32)
      def ofill(r, _):
        ones[r, pl.ds(0, LANES)] = one0
        return 0
      lax.fori_loop(0, CHUNK, ofill, 0)
    for k in range(RPT // ZR):
      roff = sid * RPT + k * ZR
      pltpu.sync_copy(zbuf_s, sh_s.at[pl.ds(roff, ZR), :])
      if with_cnt:
        pltpu.sync_copy(zbuf_c, sh_c.at[pl.ds(roff, ZR), :])
    plsc.subcore_barrier()

    def idx_copies(c, s):
      off = ebase + c * CHUNK
      return (
          pltpu.make_async_copy(
              ei_h.at[1, pl.ds(off, CHUNK)], idx_d[s], sem_ixd[s]),
          pltpu.make_async_copy(
              ei_h.at[0, pl.ds(off, CHUNK)], idx_s[s], sem_ixs[s]),
      )

    def gathers(c, s):
      cpd = pltpu.make_async_copy(pd_h.at[idx_d[s]], rows_d[s], sem_gd[s])
      cps = pltpu.make_async_copy(ps_h.at[idx_s[s]], rows_s[s], sem_gs[s])
      return cpd, cps

    def scatters_start(s):
      pltpu.async_copy(tbuf[s], sh_s.at[sidx[s]], sem_sc[s], add=True)
      if with_cnt:
        pltpu.async_copy(ones, sh_c.at[sidx[s]], sem_sc2[s], add=True)

    def scatters(s):
      out = [pltpu.make_async_copy(tbuf[s], sh_s.at[sidx[s]], sem_sc[s])]
      if with_cnt:
        out.append(
            pltpu.make_async_copy(ones, sh_c.at[sidx[s]], sem_sc2[s]))
      return out

    for s in range(NSLOT):
      for cp in idx_copies(s, s):
        cp.start()
    for s in range(GDIST):
      for cp in idx_copies(s, s):
        cp.wait()
      for cp in gathers(s, s):
        cp.start()

    def step(c, s, wait_scat, pre_idx, issue_gather):
      for cp in gathers(c, s):
        cp.wait()
      if wait_scat:
        for cp in scatters(s):
          cp.wait()
      for j in range(CHUNK // LANES):
        sidx[s][pl.ds(j * LANES, LANES)] = idx_d[s][pl.ds(j * LANES, LANES)]
      if pre_idx:
        for cp in idx_copies(c + IDIST, s):
          cp.start()
      @plsc.parallel_loop(0, CHUNK)
      def row_body(r):
        for j in range(HID // LANES):
          vd = rows_d[s][r, pl.ds(j * LANES, LANES)]
          vs = rows_s[s][r, pl.ds(j * LANES, LANES)]
          tbuf[s][r, pl.ds(j * LANES, LANES)] = _sc_tanh(vd + vs)
      scatters_start(s)
      if issue_gather:
        s2 = (s + GDIST) % NSLOT
        c2 = c + GDIST
        for cp in idx_copies(c2, s2):
          cp.wait()
        for cp in gathers(c2, s2):
          cp.start()

    for s in range(NSLOT):
      step(s, s, False, True, True)

    @pl.loop(1, NCHUNK // NSLOT - 1)
    def _(i):
      c0 = i * NSLOT
      for s in range(NSLOT):
        step(c0 + s, s, True, True, True)

    c0 = (NCHUNK // NSLOT - 1) * NSLOT
    for s in range(NSLOT):
      c = c0 + s
      step(c, s, True, c + IDIST < NCHUNK, c + GDIST < NCHUNK)
    step(NCHUNK - 1, (NCHUNK - 1) % NSLOT, True, False, False)
    for s in range(NSLOT):
      for cp in scatters(s):
        cp.wait()

    plsc.subcore_barrier()

    roff = sid * RPT
    pltpu.sync_copy(sh_s.at[pl.ds(roff, RPT), :],
                    out_s.at[cid, pl.ds(roff, RPT), :])
    if with_cnt:
      pltpu.sync_copy(sh_c.at[pl.ds(roff, RPT), :],
                      out_c.at[cid, pl.ds(roff, RPT), :])

  fn = pl.kernel(
      body, out_type=out_type, mesh=mesh,
      scratch_types=list(scratch.values()),
      compiler_params=pltpu.CompilerParams(use_tc_tiling_on_sc=False),
  )
  return fn


_sc_pass_cnt = None
_sc_pass_nocnt = None


def _get_sc_passes():
  global _sc_pass_cnt, _sc_pass_nocnt
  if _sc_pass_cnt is None:
    _sc_pass_cnt = _make_sc_edge_pass(True)
    _sc_pass_nocnt = _make_sc_edge_pass(False)
  return _sc_pass_cnt, _sc_pass_nocnt


BLK = 2000
GRID = N // BLK


def _dot(a, b):
  return lax.dot_general(a, b, (((1,), (0,)), ((), ())),
                         preferred_element_type=F32)


def _stage1_body(x_ref, encW, encb, w1a, w1b, b1, h_ref, pd_ref, ps_ref):
  h = jnp.tanh(_dot(x_ref[...], encW[...]) + encb[...])
  h_ref[...] = h
  pd_ref[...] = _dot(h, w1a[...]) + b1[...]
  ps_ref[...] = _dot(h, w1b[...])


def _gru_update(h, s0, s1, c0, c1, pd, ps, w2, b2, wih, whh, bih, bhh):
  tself = jnp.tanh(pd + ps)
  s = s0 + s1 + tself
  cnt = c0[:, :1] + c1[:, :1] + 1.0
  agg = _dot(s / cnt, w2) + b2
  gi = _dot(agg, wih) + bih
  gh = _dot(h, whh) + bhh
  r = jax.nn.sigmoid(gi[:, :H] + gh[:, :H])
  z = jax.nn.sigmoid(gi[:, H:2 * H] + gh[:, H:2 * H])
  n = jnp.tanh(gi[:, 2 * H:] + r * gh[:, 2 * H:])
  return (1.0 - z) * n + z * h


def _stage2_body(h_ref, pd_ref, ps_ref, sp, cp, w2, b2, wih, whh,
                 bih, bhh, w1a, w1b, b1, hn_ref, pdn_ref, psn_ref):
  hn = _gru_update(h_ref[...], sp[0], sp[1], cp[0], cp[1],
                   pd_ref[...], ps_ref[...], w2[...], b2[...], wih[...],
                   whh[...], bih[...], bhh[...])
  hn_ref[...] = hn
  pdn_ref[...] = _dot(hn, w1a[...]) + b1[...]
  psn_ref[...] = _dot(hn, w1b[...])


def _stage3_body(h_ref, pd_ref, ps_ref, sp, cp, w2, b2, wih, whh,
                 bih, bhh, dw1, db1, dw2, db2, out_ref):
  hn = _gru_update(h_ref[...], sp[0], sp[1], cp[0], cp[1],
                   pd_ref[...], ps_ref[...], w2[...], b2[...], wih[...],
                   whh[...], bih[...], bhh[...])
  out_ref[...] = _dot(jnp.tanh(_dot(hn, dw1[...]) + db1[...]), dw2[...]) \
      + db2[...]


def _row_spec(width):
  return pl.BlockSpec((BLK, width), lambda i: (i, 0))


def _part_spec(width):
  return pl.BlockSpec((NC, BLK, width), lambda i: (0, i, 0))


def _full_spec(shape):
  return pl.BlockSpec(shape, lambda i: tuple(0 for _ in shape))


def _tc_call(body, in_specs, out_widths, args):
  out_shape = [jax.ShapeDtypeStruct((N, w), F32) for w in out_widths]
  return pl.pallas_call(
      body,
      grid=(GRID,),
      in_specs=in_specs,
      out_specs=[_row_spec(w) for w in out_widths],
      out_shape=out_shape,
  )(*args)


def kernel(x, edge_index, enc_W, enc_b, msg_W1_0, msg_b1_0, msg_W2_0,
           msg_b2_0, gru_Wih_0, gru_Whh_0, gru_bih_0, gru_bhh_0, msg_W1_1,
           msg_b1_1, msg_W2_1, msg_b2_1, gru_Wih_1, gru_Whh_1, gru_bih_1,
           gru_bhh_1, dec_W1, dec_b1, dec_W2, dec_b2):
  enc_b2d = enc_b.reshape(1, H)
  w1a_0, w1b_0 = msg_W1_0[:H], msg_W1_0[H:]
  w1a_1, w1b_1 = msg_W1_1[:H], msg_W1_1[H:]
  b1_0 = msg_b1_0.reshape(1, HID)
  b1_1 = msg_b1_1.reshape(1, HID)
  b2_0 = msg_b2_0.reshape(1, H)
  b2_1 = msg_b2_1.reshape(1, H)
  bih_0 = gru_bih_0.reshape(1, 3 * H)
  bhh_0 = gru_bhh_0.reshape(1, 3 * H)
  bih_1 = gru_bih_1.reshape(1, 3 * H)
  bhh_1 = gru_bhh_1.reshape(1, 3 * H)
  db1 = dec_b1.reshape(1, HID)
  db2 = dec_b2.reshape(1, 1)

  sc_cnt, sc_nocnt = _get_sc_passes()

  h0, pd0, ps0 = _tc_call(
      _stage1_body,
      [_row_spec(H), _full_spec((H, H)), _full_spec((1, H)),
       _full_spec((H, HID)), _full_spec((H, HID)), _full_spec((1, HID))],
      [H, HID, HID],
      [x, enc_W, enc_b2d, w1a_0, w1b_0, b1_0],
  )

  s_part0, c_part = sc_cnt(edge_index, pd0, ps0)

  gru_specs = [_full_spec((HID, H)), _full_spec((1, H)),
               _full_spec((H, 3 * H)), _full_spec((H, 3 * H)),
               _full_spec((1, 3 * H)), _full_spec((1, 3 * H))]

  h1, pd1, ps1 = _tc_call(
      _stage2_body,
      [_row_spec(H), _row_spec(HID), _row_spec(HID), _part_spec(HID),
       _part_spec(LANES)] + gru_specs +
      [_full_spec((H, HID)), _full_spec((H, HID)), _full_spec((1, HID))],
      [H, HID, HID],
      [h0, pd0, ps0, s_part0, c_part,
       msg_W2_0, b2_0, gru_Wih_0, gru_Whh_0, bih_0, bhh_0,
       w1a_1, w1b_1, b1_1],
  )

  (s_part1,) = sc_nocnt(edge_index, pd1, ps1)

  (out,) = _tc_call(
      _stage3_body,
      [_row_spec(H), _row_spec(HID), _row_spec(HID), _part_spec(HID),
       _part_spec(LANES)] + gru_specs +
      [_full_spec((H, HID)), _full_spec((1, HID)), _full_spec((HID, 1)),
       _full_spec((1, 1))],
      [1],
      [h1, pd1, ps1, s_part1, c_part,
       msg_W2_1, b2_1, gru_Wih_1, gru_Whh_1, bih_1, bhh_1,
       dec_W1, db1, dec_W2, db2],
  )

  return out.reshape(N)
